# baseline (speedup 1.0000x reference)
import jax
import jax.numpy as jnp
from jax.experimental import pallas as pl
from jax.experimental.pallas import tpu as pltpu

_BM = 2048
_BN = 1024
_SUB = 512
_LANES = 128


def _km_kernel(f_ref, p_ref, sim_ref, ids_ref, amax_ref, aidx_ref):
    i = pl.program_id(0)
    j = pl.program_id(1)
    nj = pl.num_programs(1)

    f = f_ref[pl.ds(i * _BM, _BM), :]
    p = p_ref[pl.ds(j * _BN, _BN), :]
    fn = jnp.sqrt(jnp.sum(f * f, axis=1, keepdims=True))
    f = f / jnp.maximum(fn, 1e-12)
    pn = jnp.sqrt(jnp.sum(p * p, axis=1, keepdims=True))
    p = p / jnp.maximum(pn, 1e-12)

    @pl.when(j == 0)
    def _init():
        amax_ref[...] = jnp.full_like(amax_ref[...], -jnp.inf)
        aidx_ref[...] = jnp.zeros_like(aidx_ref[...])

    chunks = _BN // _LANES
    sub_chunks = _SUB // _LANES
    amax = amax_ref[...]
    aidx = aidx_ref[...]
    for s in range(_BN // _SUB):
        ps = p[s * _SUB:(s + 1) * _SUB, :]
        v = jax.lax.dot_general(f, ps, (((1,), (1,)), ((), ())),
                                preferred_element_type=jnp.float32)
        sim_ref[:, s * _SUB:(s + 1) * _SUB] = v
        for k in range(sub_chunks):
            vv = v[:, k * _LANES:(k + 1) * _LANES]
            chunk_id = j * chunks + s * sub_chunks + k
            gt = vv > amax
            amax = jnp.where(gt, vv, amax)
            aidx = jnp.where(gt, chunk_id, aidx)
    amax_ref[...] = amax
    aidx_ref[...] = aidx

    @pl.when(j == nj - 1)
    def _finalize():
        a = amax_ref[...]
        ai = aidx_ref[...]
        rowmax = jnp.max(a, axis=1, keepdims=True)
        lane = jax.lax.broadcasted_iota(jnp.int32, a.shape, 1)
        col = ai * _LANES + lane
        cand = jnp.where(a == rowmax, col, jnp.iinfo(jnp.int32).max)
        ids_ref[...] = jnp.min(cand, axis=1, keepdims=True)


def kernel(features, prototypes):
    m, k = features.shape
    n = prototypes.shape[0]
    sim, ids = pl.pallas_call(
        _km_kernel,
        grid=(m // _BM, n // _BN),
        in_specs=[
            pl.BlockSpec((m, k), lambda i, j: (0, 0)),
            pl.BlockSpec((n, k), lambda i, j: (0, 0)),
        ],
        out_specs=[
            pl.BlockSpec((_BM, _BN), lambda i, j: (i, j)),
            pl.BlockSpec((_BM, 1), lambda i, j: (i, 0)),
        ],
        out_shape=[
            jax.ShapeDtypeStruct((m, n), jnp.float32),
            jax.ShapeDtypeStruct((m, 1), jnp.int32),
        ],
        scratch_shapes=[
            pltpu.VMEM((_BM, _LANES), jnp.float32),
            pltpu.VMEM((_BM, _LANES), jnp.int32),
        ],
        compiler_params=pltpu.CompilerParams(
            dimension_semantics=("parallel", "arbitrary"),
        ),
    )(features, prototypes)
    return ids.reshape(m), sim


# speedup vs baseline: 1.0677x; 1.0677x over previous
import jax
import jax.numpy as jnp
from jax.experimental import pallas as pl
from jax.experimental.pallas import tpu as pltpu

_BM = 2048
_BN = 1024
_SUB = 512
_LANES = 128


def _km_kernel(f_ref, p_ref, sim_ref, ids_ref, amax_ref, aidx_ref,
               fhat_ref, phat_ref):
    i = pl.program_id(0)
    j = pl.program_id(1)
    nj = pl.num_programs(1)

    @pl.when(j == 0)
    def _norm_f():
        f = f_ref[pl.ds(i * _BM, _BM), :]
        fn = jnp.sqrt(jnp.sum(f * f, axis=1, keepdims=True))
        fhat_ref[...] = f / jnp.maximum(fn, 1e-12)
        amax_ref[...] = jnp.full_like(amax_ref[...], -jnp.inf)
        aidx_ref[...] = jnp.zeros_like(aidx_ref[...])

    @pl.when(i == 0)
    def _norm_p():
        p = p_ref[pl.ds(j * _BN, _BN), :]
        pn = jnp.sqrt(jnp.sum(p * p, axis=1, keepdims=True))
        phat_ref[pl.ds(j * _BN, _BN), :] = p / jnp.maximum(pn, 1e-12)

    f = fhat_ref[...]
    chunks = _BN // _LANES
    sub_chunks = _SUB // _LANES
    amax = amax_ref[...]
    aidx = aidx_ref[...]
    for s in range(_BN // _SUB):
        ps = phat_ref[pl.ds(j * _BN + s * _SUB, _SUB), :]
        v = jax.lax.dot_general(f, ps, (((1,), (1,)), ((), ())),
                                preferred_element_type=jnp.float32)
        sim_ref[:, s * _SUB:(s + 1) * _SUB] = v
        for k in range(sub_chunks):
            vv = v[:, k * _LANES:(k + 1) * _LANES]
            chunk_id = j * chunks + s * sub_chunks + k
            gt = vv > amax
            amax = jnp.where(gt, vv, amax)
            aidx = jnp.where(gt, chunk_id, aidx)
    amax_ref[...] = amax
    aidx_ref[...] = aidx

    @pl.when(j == nj - 1)
    def _finalize():
        a = amax_ref[...]
        ai = aidx_ref[...]
        rowmax = jnp.max(a, axis=1, keepdims=True)
        lane = jax.lax.broadcasted_iota(jnp.int32, a.shape, 1)
        col = ai * _LANES + lane
        cand = jnp.where(a == rowmax, col, jnp.iinfo(jnp.int32).max)
        ids_ref[...] = jnp.min(cand, axis=1, keepdims=True)


def kernel(features, prototypes):
    m, k = features.shape
    n = prototypes.shape[0]
    sim, ids = pl.pallas_call(
        _km_kernel,
        grid=(m // _BM, n // _BN),
        in_specs=[
            pl.BlockSpec((m, k), lambda i, j: (0, 0)),
            pl.BlockSpec((n, k), lambda i, j: (0, 0)),
        ],
        out_specs=[
            pl.BlockSpec((_BM, _BN), lambda i, j: (i, j)),
            pl.BlockSpec((_BM, 1), lambda i, j: (i, 0)),
        ],
        out_shape=[
            jax.ShapeDtypeStruct((m, n), jnp.float32),
            jax.ShapeDtypeStruct((m, 1), jnp.int32),
        ],
        scratch_shapes=[
            pltpu.VMEM((_BM, _LANES), jnp.float32),
            pltpu.VMEM((_BM, _LANES), jnp.int32),
            pltpu.VMEM((_BM, k), jnp.float32),
            pltpu.VMEM((n, k), jnp.float32),
        ],
        compiler_params=pltpu.CompilerParams(
            dimension_semantics=("parallel", "arbitrary"),
        ),
    )(features, prototypes)
    return ids.reshape(m), sim
